# Initial kernel scaffold; baseline (speedup 1.0000x reference)
import jax
import jax.numpy as jnp
from jax.experimental import pallas as pl


def kernel(input_ids, attention_mask, W):
    raise NotImplementedError("write your pallas kernel here")


# speedup vs baseline: 1.9595x; 1.9595x over previous
import jax
import jax.numpy as jnp
from jax import lax
from jax.experimental import pallas as pl
from jax.experimental.pallas import tpu as pltpu
from jax.experimental.pallas import tpu_sc as plsc

B, S, D = 1024, 200, 128
L = 16
NC, NS = 2, 16
NW = NC * NS
RW = B // NW
SP = 208
HC = SP // 2
NG = SP // L
NCH = D // L


def _body(ids_hbm, mask_hbm, w_hbm, out_hbm,
          ids_v, mask_v, maskf_v, rows_v, outrow_v, sem0, sem1):
    wid = lax.axis_index("s") * NC + lax.axis_index("c")
    base = wid * RW

    def row_body(r, carry):
        b = base + r
        ids_v[pl.ds(SP - L, L)] = jnp.zeros((L,), jnp.int32)
        mask_v[pl.ds(SP - L, L)] = jnp.zeros((L,), jnp.int32)
        pltpu.sync_copy(ids_hbm.at[pl.ds(b * S, S)], ids_v.at[pl.ds(0, S)])
        pltpu.sync_copy(mask_hbm.at[pl.ds(b * S, S)], mask_v.at[pl.ds(0, S)])
        cp0 = pltpu.async_copy(w_hbm.at[ids_v.at[pl.ds(0, HC)]],
                               rows_v.at[pl.ds(0, HC)], sem0)
        cp1 = pltpu.async_copy(w_hbm.at[ids_v.at[pl.ds(HC, HC)]],
                               rows_v.at[pl.ds(HC, HC)], sem1)

        def cnt_body(j, acc):
            mf = mask_v[pl.ds(j * L, L)].astype(jnp.float32)
            maskf_v[pl.ds(j * L, L)] = mf
            return acc + mf
        cntv = lax.fori_loop(0, NG, cnt_body, jnp.zeros((L,), jnp.float32))
        cnt = cntv[0]
        for k in range(1, L):
            cnt = cnt + cntv[k]
        inv = jnp.ones((L,), jnp.float32) / jnp.maximum(
            jnp.full((L,), cnt, jnp.float32), 1.0)

        cp0.wait()
        cp1.wait()

        def acc_body(g, a):
            mvec = maskf_v[pl.ds(g * L, L)]
            t0 = g * L
            for k in range(L):
                m = mvec[k]
                a = tuple(a[c] + rows_v[t0 + k, pl.ds(c * L, L)] * m
                          for c in range(NCH))
            return a
        acc = lax.fori_loop(0, NG, acc_body,
                            (jnp.zeros((L,), jnp.float32),) * NCH)

        for c in range(NCH):
            outrow_v[pl.ds(c * L, L)] = acc[c] * inv
        pltpu.sync_copy(outrow_v, out_hbm.at[pl.ds(b * D, D)])
        return carry

    lax.fori_loop(0, RW, row_body, 0)


def kernel(input_ids, attention_mask, W):
    mesh = plsc.VectorSubcoreMesh(core_axis_name="c", subcore_axis_name="s")
    k = pl.kernel(
        _body,
        out_type=jax.ShapeDtypeStruct((B * D,), jnp.float32),
        mesh=mesh,
        scratch_types=[
            pltpu.VMEM((SP,), jnp.int32),
            pltpu.VMEM((SP,), jnp.int32),
            pltpu.VMEM((SP,), jnp.float32),
            pltpu.VMEM((SP, D), jnp.float32),
            pltpu.VMEM((D,), jnp.float32),
            pltpu.SemaphoreType.DMA,
            pltpu.SemaphoreType.DMA,
        ],
    )
    out = k(input_ids.astype(jnp.int32).reshape(-1),
            attention_mask.astype(jnp.int32).reshape(-1), W)
    return out.reshape(B, D)


# speedup vs baseline: 9.5832x; 4.8907x over previous
import jax
import jax.numpy as jnp
from jax import lax
from jax.experimental import pallas as pl
from jax.experimental.pallas import tpu as pltpu
from jax.experimental.pallas import tpu_sc as plsc

B, S, D = 1024, 200, 128
L = 16
NC, NS = 2, 16
NW = NC * NS
RW = B // NW
HC0, HC1 = 104, 96
NG = S // L
TAIL = S - NG * L
NCH = D // L


def _body(ids_hbm, mask_hbm, w_hbm, out_hbm,
          ids_all, mask_all, maskf_v, rows0, rows1, out_all,
          si0, si1, sj0, sj1):
    wid = lax.axis_index("s") * NC + lax.axis_index("c")
    base = wid * RW
    pltpu.sync_copy(ids_hbm.at[pl.ds(base * S, RW * S)], ids_all)
    pltpu.sync_copy(mask_hbm.at[pl.ds(base * S, RW * S)], mask_all)

    def gathers(r, buf, s0, s1):
        off = r * S
        return (
            pltpu.make_async_copy(w_hbm.at[ids_all.at[pl.ds(off, HC0)]],
                                  buf.at[pl.ds(0, HC0)], s0),
            pltpu.make_async_copy(w_hbm.at[ids_all.at[pl.ds(off + HC0, HC1)]],
                                  buf.at[pl.ds(HC0, HC1)], s1),
        )

    def fire(r, buf, s0, s1):
        g0, g1 = gathers(r, buf, s0, s1)
        g0.start()
        g1.start()

    def process(r, buf, s0, s1):
        off = r * S

        def cnt_body(g, acc):
            mf = mask_all[pl.ds(off + g * L, L)].astype(jnp.float32)
            maskf_v[pl.ds(g * L, L)] = mf
            return acc + mf
        cntv = lax.fori_loop(0, NG, cnt_body, jnp.zeros((L,), jnp.float32))
        tailm = mask_all[pl.ds(off + S - L, L)].astype(jnp.float32)
        cnt = cntv[0]
        for k in range(1, L):
            cnt = cnt + cntv[k]
        for k in range(L - TAIL, L):
            cnt = cnt + tailm[k]
        inv = jnp.ones((L,), jnp.float32) / jnp.maximum(
            jnp.full((L,), cnt, jnp.float32), 1.0)

        g0, g1 = gathers(r, buf, s0, s1)
        g0.wait()
        g1.wait()

        def acc_body(g, a):
            mvec = maskf_v[pl.ds(g * L, L)]
            t0 = g * L
            for k in range(L):
                m = mvec[k]
                a = tuple(a[c] + buf[t0 + k, pl.ds(c * L, L)] * m
                          for c in range(NCH))
            return a
        acc = lax.fori_loop(0, NG, acc_body,
                            (jnp.zeros((L,), jnp.float32),) * NCH)
        for k in range(TAIL):
            m = tailm[L - TAIL + k]
            acc = tuple(acc[c] + buf[NG * L + k, pl.ds(c * L, L)] * m
                        for c in range(NCH))

        for c in range(NCH):
            out_all[pl.ds(r * D + c * L, L)] = acc[c] * inv

    fire(0, rows0, si0, si1)

    def iter_body(i, carry):
        r0 = 2 * i
        fire(r0 + 1, rows1, sj0, sj1)
        process(r0, rows0, si0, si1)

        @pl.when(i < RW // 2 - 1)
        def _():
            fire(r0 + 2, rows0, si0, si1)
        process(r0 + 1, rows1, sj0, sj1)
        return carry

    lax.fori_loop(0, RW // 2, iter_body, 0)
    pltpu.sync_copy(out_all, out_hbm.at[pl.ds(base * D, RW * D)])


def kernel(input_ids, attention_mask, W):
    mesh = plsc.VectorSubcoreMesh(core_axis_name="c", subcore_axis_name="s")
    k = pl.kernel(
        _body,
        out_type=jax.ShapeDtypeStruct((B * D,), jnp.float32),
        mesh=mesh,
        scratch_types=[
            pltpu.VMEM((RW * S,), jnp.int32),
            pltpu.VMEM((RW * S,), jnp.int32),
            pltpu.VMEM((NG * L,), jnp.float32),
            pltpu.VMEM((S, D), jnp.float32),
            pltpu.VMEM((S, D), jnp.float32),
            pltpu.VMEM((RW * D,), jnp.float32),
            pltpu.SemaphoreType.DMA,
            pltpu.SemaphoreType.DMA,
            pltpu.SemaphoreType.DMA,
            pltpu.SemaphoreType.DMA,
        ],
    )
    out = k(input_ids.astype(jnp.int32).reshape(-1),
            attention_mask.astype(jnp.int32).reshape(-1), W)
    return out.reshape(B, D)
